# baseline (speedup 1.0000x reference)
import functools

import jax
import jax.numpy as jnp
import numpy as np
from jax import lax
from jax.experimental import pallas as pl
from jax.experimental.pallas import tpu as pltpu
from jax.experimental.pallas import tpu_sc as plsc

PW = 0.1
NGRAM = 4
_SC_PARAMS = pltpu.CompilerParams(needs_layout_passes=False)


def _sim_body(h_ref, p_ref, w_ref, out_ref):
    dn = (((1,), (1,)), ((), ()))
    h1 = lax.dot_general(h_ref[...], w_ref[...], dn,
                         preferred_element_type=jnp.float32)
    h2 = lax.dot_general(p_ref[...], w_ref[...], dn,
                         preferred_element_type=jnp.float32)
    dot = jnp.sum(h1 * h2, axis=-1)
    n1 = jnp.maximum(jnp.sqrt(jnp.sum(h1 * h1, axis=-1)), 1e-8)
    n2 = jnp.maximum(jnp.sqrt(jnp.sum(h2 * h2, axis=-1)), 1e-8)
    sim_pen = jnp.clip(dot / (n1 * n2), 0.0, None) * PW
    out_ref[...] = sim_pen[None, :]


def _sim_call(hidden, prev, w):
    b = hidden.shape[0]
    return pl.pallas_call(
        _sim_body,
        out_shape=jax.ShapeDtypeStruct((1, b), jnp.float32),
    )(hidden, prev, w)


def _bias_body(l_ref, b_ref, o_ref):
    o_ref[...] = l_ref[...] + b_ref[...]


def _bias_call(logits, bias):
    b, v = logits.shape
    blk = 12800
    grid = (v + blk - 1) // blk
    return pl.pallas_call(
        _bias_body,
        grid=(grid,),
        in_specs=[pl.BlockSpec((b, blk), lambda i: (0, i)),
                  pl.BlockSpec((1, blk), lambda i: (0, i))],
        out_specs=pl.BlockSpec((b, blk), lambda i: (0, i)),
        out_shape=jax.ShapeDtypeStruct((b, v), jnp.float32),
    )(logits, bias.reshape(1, v))


def _sc_call(out0, gen, sim):
    b, v = out0.shape
    s = gen.shape[1]
    wn = s - NGRAM + 1
    nv = s // 16
    padw = s
    sent = s
    histw = ((s + 1 + 15) // 16) * 16
    rows_per_w = b // 32
    c = float(np.float32(PW) / np.float32(s))
    mesh = plsc.VectorSubcoreMesh(core_axis_name="c", subcore_axis_name="s")

    @functools.partial(
        pl.kernel,
        out_type=(jax.ShapeDtypeStruct((b, v), jnp.float32),
                  jax.ShapeDtypeStruct((b, 16), jnp.float32)),
        mesh=mesh,
        compiler_params=_SC_PARAMS,
        scratch_types=[
            pltpu.VMEM((v,), jnp.float32),
            pltpu.VMEM((s,), jnp.int32),
            pltpu.VMEM((s,), jnp.int32),
            pltpu.VMEM((s + 16,), jnp.int32),
            pltpu.VMEM((s + 16,), jnp.int32),
            pltpu.VMEM((s + 16,), jnp.int32),
            pltpu.VMEM((s + 16,), jnp.int32),
            pltpu.VMEM((histw,), jnp.int32),
            pltpu.VMEM((16,), jnp.float32),
            pltpu.VMEM((16,), jnp.float32),
        ],
    )
    def sc_kernel(out0_hbm, gen_hbm, sim_hbm, out_hbm, temp_hbm, buf,
                  tok0, tok1, tids, dpos, ws_a, ws_b, hist, simv, out16):
        wid = lax.axis_index("s") * 2 + lax.axis_index("c")
        lanes = lax.broadcasted_iota(jnp.int32, (16,), 0)
        lanes_f = lanes.astype(jnp.float32)
        dpos[pl.ds(s, 16)] = jnp.zeros((16,), jnp.int32)
        toks = [tok0, tok1]

        def radix_distinct(tok):
            def build_table(i, cc):
                t = tok[pl.ds(i * 16, 16)]
                _, last = plsc.scan_count(t)
                plsc.store_scatter(buf, [t], (i * 16 + lanes_f), mask=last)
                return cc
            lax.fori_loop(0, nv, build_table, 0)

            def fill_ids(i, cc):
                t = tok[pl.ds(i * 16, 16)]
                tids[pl.ds(i * 16, 16)] = plsc.load_gather(
                    buf, [t]).astype(jnp.int32)
                return cc
            lax.fori_loop(0, nv, fill_ids, 0)
            tids[pl.ds(s, 16)] = jnp.full((16,), sent, jnp.int32)

            def init_order(i, cc):
                vals = i * 16 + lanes
                ws_a[pl.ds(i * 16, 16)] = jnp.where(vals < wn, vals, padw)
                return cc
            lax.fori_loop(0, nv, init_order, 0)

            for p in range(NGRAM):
                src, dst = (ws_a, ws_b) if p % 2 == 0 else (ws_b, ws_a)
                koff = NGRAM - 1 - p

                def zero_hist(i, cc):
                    hist[pl.ds(i * 16, 16)] = jnp.zeros((16,), jnp.int32)
                    return cc
                lax.fori_loop(0, histw // 16, zero_hist, 0)

                def histo(i, cc):
                    w = src[pl.ds(i * 16, 16)]
                    d = plsc.load_gather(tids, [w + koff])
                    occ, last = plsc.scan_count(d)
                    plsc.addupdate_scatter(hist, [d], occ, mask=last)
                    return cc
                lax.fori_loop(0, nv, histo, 0)

                def excl_scan(i, run):
                    h = hist[pl.ds(i * 16, 16)]
                    csum = plsc.cumsum(h)
                    hist[pl.ds(i * 16, 16)] = csum - h + run
                    return run + jnp.sum(h)
                lax.fori_loop(0, histw // 16, excl_scan, jnp.int32(0))

                def permute(i, cc):
                    w = src[pl.ds(i * 16, 16)]
                    d = plsc.load_gather(tids, [w + koff])
                    base = plsc.load_gather(hist, [d])
                    occ, last = plsc.scan_count(d)
                    plsc.store_scatter(dst, [base + occ - 1], w)
                    plsc.addupdate_scatter(hist, [d], occ, mask=last)
                    return cc
                lax.fori_loop(0, nv, permute, 0)

            final = ws_a if NGRAM % 2 == 0 else ws_b
            final[pl.ds(s, 16)] = jnp.full((16,), padw, jnp.int32)

            def count(i, acc):
                a = final[pl.ds(i * 16, 16)]
                bb = final[pl.ds(i * 16 + 1, 16)]
                neq = jnp.zeros((16,), jnp.bool_)
                for k in range(NGRAM):
                    da = plsc.load_gather(tids, [a + k])
                    db = plsc.load_gather(tids, [bb + k])
                    neq = neq | (da != db)
                return acc + plsc.all_reduce_population_count(neq)
            return lax.fori_loop(0, nv, count, jnp.zeros((16,), jnp.int32))

        for r in range(rows_per_w):
            row = wid * rows_per_w + r
            tok = toks[r]
            pltpu.sync_copy(gen_hbm.at[row], tok)

            def build_last(i, cc):
                t = tok[pl.ds(i * 16, 16)]
                _, last = plsc.scan_count(t)
                plsc.store_scatter(buf, [t], i * 16 + lanes_f, mask=last)
                return cc
            lax.fori_loop(0, nv, build_last, 0)

            @plsc.parallel_loop(0, nv, unroll=4)
            def flag_non_last(i):
                t = tok[pl.ds(i * 16, 16)]
                lp = plsc.load_gather(buf, [t])
                dpos[pl.ds(i * 16, 16)] = (
                    lp != i * 16 + lanes_f).astype(jnp.int32)

            def build_first(i, cc):
                j = nv - 1 - i
                t = tok[pl.ds(j * 16, 16)]
                tr = lax.rev(t, (0,))
                _, firstm = plsc.scan_count(tr)
                plsc.store_scatter(buf, [tr], j * 16 + (15.0 - lanes_f),
                                   mask=firstm)
                return cc
            lax.fori_loop(0, nv, build_first, 0)

            @plsc.parallel_loop(0, nv, unroll=4)
            def flag_non_first(i):
                t = tok[pl.ds(i * 16, 16)]
                fp = plsc.load_gather(buf, [t])
                dpos[pl.ds(i * 16, 16)] = dpos[pl.ds(i * 16, 16)] | (
                    fp != i * 16 + lanes_f).astype(jnp.int32)

            @plsc.parallel_loop(0, nv, unroll=4, carry=jnp.int32(0))
            def ncand(i, acc):
                f = dpos[pl.ds(i * 16, 16)]
                for k in range(1, NGRAM):
                    f = f & dpos[pl.ds(i * 16 + k, 16)]
                valid = (i * 16 + lanes) < wn
                return acc + jnp.sum(jnp.where(valid, f, 0))

            uniq = lax.cond(
                ncand == 0,
                lambda: jnp.full((16,), wn, jnp.int32),
                lambda: radix_distinct(tok))

            pltpu.sync_copy(sim_hbm.at[row // 16], simv)
            sim_row = jnp.sum(jnp.where(lanes == row % 16, simv[...], 0.0))
            rep = (wn - uniq.astype(jnp.float32)) / float(wn)
            out16[...] = 1.0 + (rep + sim_row) * 0.5
            pltpu.sync_copy(out16, temp_hbm.at[row])

        for r in range(rows_per_w):
            row = wid * rows_per_w + r
            tok = toks[r]
            pltpu.sync_copy(out0_hbm.at[row], buf)

            @plsc.parallel_loop(0, nv, unroll=4)
            def scat(i):
                idx = tok[pl.ds(i * 16, 16)]
                cnt, last = plsc.scan_count(idx)
                plsc.addupdate_scatter(
                    buf, [idx], cnt.astype(jnp.float32) * (-c), mask=last)

            pltpu.sync_copy(buf, out_hbm.at[row])

    return sc_kernel(out0, gen, sim)


def kernel(logits, hidden_state, prev_hidden, generated_ids, W, bias):
    b, v = logits.shape
    sim = _sim_call(hidden_state, prev_hidden, W).reshape(b // 16, 16)
    out0 = _bias_call(logits, bias)
    out, temp16 = _sc_call(out0, generated_ids, sim)
    return out, temp16[:, 0]


# speedup vs baseline: 1.3519x; 1.3519x over previous
import functools

import jax
import jax.numpy as jnp
import numpy as np
from jax import lax
from jax.experimental import pallas as pl
from jax.experimental.pallas import tpu as pltpu
from jax.experimental.pallas import tpu_sc as plsc

PW = 0.1
NGRAM = 4
_SC_PARAMS = pltpu.CompilerParams(needs_layout_passes=False)


def _ngram_call(gen, v):
    b, s = gen.shape
    wn = s - NGRAM + 1
    nv = s // 16
    padw = s
    sent = s
    nbins = s + 1
    histw = ((nbins + 15) // 16) * 16
    rows_per_w = b // 32
    mesh = plsc.VectorSubcoreMesh(core_axis_name="c", subcore_axis_name="s")

    @functools.partial(
        pl.kernel,
        out_type=jax.ShapeDtypeStruct((b, 16), jnp.int32),
        mesh=mesh,
        compiler_params=_SC_PARAMS,
        scratch_types=[
            pltpu.VMEM((v,), jnp.int32),
            pltpu.VMEM((s,), jnp.int32),
            pltpu.VMEM((s + 16,), jnp.int32),
            pltpu.VMEM((s + 16,), jnp.int32),
            pltpu.VMEM((s + 16,), jnp.int32),
            pltpu.VMEM((s + 16,), jnp.int32),
            pltpu.VMEM((histw,), jnp.int32),
            pltpu.VMEM((16,), jnp.int32),
        ],
    )
    def ngram_kernel(gen_hbm, cnt_hbm, table, tok, tids, dpos, ws_a, ws_b,
                     hist, out16):
        wid = lax.axis_index("s") * 2 + lax.axis_index("c")
        lanes = lax.broadcasted_iota(jnp.int32, (16,), 0)
        dpos[pl.ds(s, 16)] = jnp.zeros((16,), jnp.int32)

        def radix_distinct():
            def build_table(i, c):
                t = tok[pl.ds(i * 16, 16)]
                _, last = plsc.scan_count(t)
                plsc.store_scatter(table, [t], i * 16 + lanes, mask=last)
                return c
            lax.fori_loop(0, nv, build_table, 0)

            def fill_ids(i, c):
                t = tok[pl.ds(i * 16, 16)]
                tids[pl.ds(i * 16, 16)] = plsc.load_gather(table, [t])
                return c
            lax.fori_loop(0, nv, fill_ids, 0)
            tids[pl.ds(s, 16)] = jnp.full((16,), sent, jnp.int32)

            def init_order(i, c):
                vals = i * 16 + lanes
                ws_a[pl.ds(i * 16, 16)] = jnp.where(vals < wn, vals, padw)
                return c
            lax.fori_loop(0, nv, init_order, 0)

            for p in range(NGRAM):
                src, dst = (ws_a, ws_b) if p % 2 == 0 else (ws_b, ws_a)
                koff = NGRAM - 1 - p

                def zero_hist(i, c):
                    hist[pl.ds(i * 16, 16)] = jnp.zeros((16,), jnp.int32)
                    return c
                lax.fori_loop(0, histw // 16, zero_hist, 0)

                def histo(i, c):
                    w = src[pl.ds(i * 16, 16)]
                    d = plsc.load_gather(tids, [w + koff])
                    occ, last = plsc.scan_count(d)
                    plsc.addupdate_scatter(hist, [d], occ, mask=last)
                    return c
                lax.fori_loop(0, nv, histo, 0)

                def excl_scan(i, run):
                    h = hist[pl.ds(i * 16, 16)]
                    csum = plsc.cumsum(h)
                    hist[pl.ds(i * 16, 16)] = csum - h + run
                    return run + jnp.sum(h)
                lax.fori_loop(0, histw // 16, excl_scan, jnp.int32(0))

                def permute(i, c):
                    w = src[pl.ds(i * 16, 16)]
                    d = plsc.load_gather(tids, [w + koff])
                    base = plsc.load_gather(hist, [d])
                    occ, last = plsc.scan_count(d)
                    plsc.store_scatter(dst, [base + occ - 1], w)
                    plsc.addupdate_scatter(hist, [d], occ, mask=last)
                    return c
                lax.fori_loop(0, nv, permute, 0)

            final = ws_a if NGRAM % 2 == 0 else ws_b
            final[pl.ds(s, 16)] = jnp.full((16,), padw, jnp.int32)

            def count(i, acc):
                a = final[pl.ds(i * 16, 16)]
                bb = final[pl.ds(i * 16 + 1, 16)]
                neq = jnp.zeros((16,), jnp.bool_)
                for k in range(NGRAM):
                    da = plsc.load_gather(tids, [a + k])
                    db = plsc.load_gather(tids, [bb + k])
                    neq = neq | (da != db)
                return acc + plsc.all_reduce_population_count(neq)
            return lax.fori_loop(0, nv, count, jnp.zeros((16,), jnp.int32))

        for r in range(rows_per_w):
            row = wid * rows_per_w + r
            pltpu.sync_copy(gen_hbm.at[row], tok)

            def build_last(i, c):
                t = tok[pl.ds(i * 16, 16)]
                _, last = plsc.scan_count(t)
                plsc.store_scatter(table, [t], i * 16 + lanes, mask=last)
                return c
            lax.fori_loop(0, nv, build_last, 0)

            @plsc.parallel_loop(0, nv, unroll=4)
            def flag_non_last(i):
                t = tok[pl.ds(i * 16, 16)]
                lp = plsc.load_gather(table, [t])
                dpos[pl.ds(i * 16, 16)] = (
                    lp != i * 16 + lanes).astype(jnp.int32)

            def build_first(i, c):
                j = nv - 1 - i
                t = tok[pl.ds(j * 16, 16)]
                tr = lax.rev(t, (0,))
                _, firstm = plsc.scan_count(tr)
                plsc.store_scatter(table, [tr], j * 16 + (15 - lanes),
                                   mask=firstm)
                return c
            lax.fori_loop(0, nv, build_first, 0)

            @plsc.parallel_loop(0, nv, unroll=4)
            def flag_non_first(i):
                t = tok[pl.ds(i * 16, 16)]
                fp = plsc.load_gather(table, [t])
                dpos[pl.ds(i * 16, 16)] = dpos[pl.ds(i * 16, 16)] | (
                    fp != i * 16 + lanes).astype(jnp.int32)

            @plsc.parallel_loop(0, nv, unroll=4, carry=jnp.int32(0))
            def ncand(i, acc):
                f = dpos[pl.ds(i * 16, 16)]
                for k in range(1, NGRAM):
                    f = f & dpos[pl.ds(i * 16 + k, 16)]
                valid = (i * 16 + lanes) < wn
                return acc + jnp.sum(jnp.where(valid, f, 0))

            acc = lax.cond(
                ncand == 0,
                lambda: jnp.full((16,), wn, jnp.int32),
                radix_distinct)

            out16[...] = acc
            pltpu.sync_copy(out16, cnt_hbm.at[row])

    return ngram_kernel(gen)


def _sim_temp_body(h_ref, p_ref, w_ref, cnt_ref, out_ref, *, wn):
    dn = (((1,), (1,)), ((), ()))
    h1 = lax.dot_general(h_ref[...], w_ref[...], dn,
                         preferred_element_type=jnp.float32)
    h2 = lax.dot_general(p_ref[...], w_ref[...], dn,
                         preferred_element_type=jnp.float32)
    dot = jnp.sum(h1 * h2, axis=-1)
    n1 = jnp.maximum(jnp.sqrt(jnp.sum(h1 * h1, axis=-1)), 1e-8)
    n2 = jnp.maximum(jnp.sqrt(jnp.sum(h2 * h2, axis=-1)), 1e-8)
    sim_pen = jnp.clip(dot / (n1 * n2), 0.0, None) * PW
    uniq = cnt_ref[...][:, 0].astype(jnp.float32)
    rep = (wn - uniq) / wn
    out_ref[...] = (1.0 + (rep + sim_pen) * 0.5)[None, :]


def _sim_temp_call(hidden, prev, w, cnts, wn):
    b = hidden.shape[0]
    return pl.pallas_call(
        functools.partial(_sim_temp_body, wn=float(wn)),
        out_shape=jax.ShapeDtypeStruct((1, b), jnp.float32),
    )(hidden, prev, w, cnts)


def _bias_body(l_ref, b_ref, o_ref):
    o_ref[...] = l_ref[...] + b_ref[...]


def _bias_call(logits, bias):
    b, v = logits.shape
    blk = 12800
    grid = (v + blk - 1) // blk
    return pl.pallas_call(
        _bias_body,
        grid=(grid,),
        in_specs=[pl.BlockSpec((b, blk), lambda i: (0, i)),
                  pl.BlockSpec((1, blk), lambda i: (0, i))],
        out_specs=pl.BlockSpec((b, blk), lambda i: (0, i)),
        out_shape=jax.ShapeDtypeStruct((b, v), jnp.float32),
    )(logits, bias.reshape(1, v))


def _freq_call(out0, gen):
    b, v = out0.shape
    s = gen.shape[1]
    lanes = 16
    rows_per_w = b // 32
    c = float(np.float32(PW) / np.float32(s))
    mesh = plsc.VectorSubcoreMesh(core_axis_name="c", subcore_axis_name="s")

    @functools.partial(
        pl.kernel,
        out_type=jax.ShapeDtypeStruct((b, v), jnp.float32),
        mesh=mesh,
        compiler_params=_SC_PARAMS,
        scratch_types=[
            pltpu.VMEM((v,), jnp.float32),
            pltpu.VMEM((s,), jnp.int32),
        ],
    )
    def freq_kernel(out0_hbm, gen_hbm, out_hbm, buf, tok):
        wid = lax.axis_index("s") * 2 + lax.axis_index("c")
        for r in range(rows_per_w):
            row = wid * rows_per_w + r
            pltpu.sync_copy(out0_hbm.at[row], buf)
            pltpu.sync_copy(gen_hbm.at[row], tok)

            @plsc.parallel_loop(0, s // lanes, unroll=4)
            def body(i):
                idx = tok[pl.ds(i * lanes, lanes)]
                cnt, last = plsc.scan_count(idx)
                plsc.addupdate_scatter(
                    buf, [idx], cnt.astype(jnp.float32) * (-c), mask=last)
            pltpu.sync_copy(buf, out_hbm.at[row])

    return freq_kernel(out0, gen)


def kernel(logits, hidden_state, prev_hidden, generated_ids, W, bias):
    b, v = logits.shape
    s = generated_ids.shape[1]
    wn = s - NGRAM + 1

    cnts = _ngram_call(generated_ids, v)
    out0 = _bias_call(logits, bias)
    out = _freq_call(out0, generated_ids)
    temp = _sim_temp_call(hidden_state, prev_hidden, W, cnts, wn)
    return out, temp.reshape(b)
